# Initial kernel scaffold; baseline (speedup 1.0000x reference)
import functools

import jax
import jax.numpy as jnp
from jax import lax
from jax.experimental import pallas as pl
from jax.experimental.pallas import tpu as pltpu
from jax.experimental.pallas import tpu_sc as plsc

_N = 10000
_E = 320000
_G = 64
_CH = 128
_NCH = _E // _CH
_NTILES = 16
_BASE_CH = _NCH // _NTILES
_EXTRA_CH = _NCH % _NTILES
_RPT = _N // _NTILES
_F32 = jnp.float32


def _sc_edge_body(h_lo, h_hi, e_lo, e_hi, src2, dst2, agg_lo, agg_hi,
                  acc, sidx, didx, rows, ev, sem):
    cid = lax.axis_index("c")
    sid = lax.axis_index("s")

    def zrow(r, carry):
        for c8 in range(8):
            rows[r, pl.ds(c8 * 16, 16)] = jnp.zeros((16,), _F32)
        return carry
    lax.fori_loop(0, _CH, zrow, 0)
    base0 = sid * _RPT
    off = 0
    while off < _RPT:
        nr = min(_CH, _RPT - off)
        pltpu.sync_copy(rows.at[pl.ds(0, nr)], acc.at[pl.ds(base0 + off, nr)])
        off += nr
    plsc.subcore_barrier()

    nch = jnp.where(sid < _EXTRA_CH, _BASE_CH + 1, _BASE_CH)

    def make_work(h_ref, e_ref):
        def work():
            def body(i, carry):
                chunk = sid + _NTILES * i
                pltpu.sync_copy(src2.at[pl.ds(chunk, 1)], sidx)
                pltpu.sync_copy(dst2.at[pl.ds(chunk, 1)], didx)
                gat = pltpu.async_copy(h_ref.at[sidx.at[0]], rows, sem)
                pltpu.sync_copy(e_ref.at[pl.ds(chunk * _CH, _CH)], ev)
                gat.wait()

                def crow(r, cc):
                    for c8 in range(8):
                        sl = pl.ds(c8 * 16, 16)
                        ev[r, sl] = jnp.maximum(rows[r, sl] + ev[r, sl], 0.0)
                    return cc
                lax.fori_loop(0, _CH, crow, 0)
                pltpu.sync_copy(ev, acc.at[didx.at[0]], add=True)
                return carry
            lax.fori_loop(0, nch, body, 0)
        return work

    pl.when(cid == 0)(make_work(h_lo, e_lo))
    pl.when(cid == 1)(make_work(h_hi, e_hi))
    plsc.subcore_barrier()

    def make_flush(o_ref):
        def flush():
            pltpu.sync_copy(acc.at[pl.ds(base0, _RPT)],
                            o_ref.at[pl.ds(base0, _RPT)])
        return flush

    pl.when(cid == 0)(make_flush(agg_lo))
    pl.when(cid == 1)(make_flush(agg_hi))


def _sc_edge(h_lo, h_hi, e_lo, e_hi, src2, dst2):
    f = pl.kernel(
        _sc_edge_body,
        out_type=[jax.ShapeDtypeStruct((_N, 128), _F32),
                  jax.ShapeDtypeStruct((_N, 128), _F32)],
        mesh=plsc.VectorSubcoreMesh(core_axis_name="c", subcore_axis_name="s"),
        scratch_types=[
            pltpu.VMEM_SHARED((_N, 128), _F32),
            pltpu.VMEM((1, _CH), jnp.int32),
            pltpu.VMEM((1, _CH), jnp.int32),
            pltpu.VMEM((_CH, 128), _F32),
            pltpu.VMEM((_CH, 128), _F32),
            pltpu.SemaphoreType.DMA,
        ],
    )
    return f(h_lo, h_hi, e_lo, e_hi, src2, dst2)


def _enc_h_body(x_ref, w_ref, b_ref, lo_ref, hi_ref):
    h = jnp.dot(x_ref[...], w_ref[...], preferred_element_type=_F32) + b_ref[...]
    lo_ref[...] = h[:, :128]
    hi_ref[...] = h[:, 128:]


def _enc_e_body(ea_ref, w_ref, b_ref, lo_ref, hi_ref):
    v = jnp.dot(ea_ref[...], w_ref[...], preferred_element_type=_F32) + b_ref[...]
    lo_ref[...] = v[:, :128]
    hi_ref[...] = v[:, 128:]


def _layer_body(hlo, hhi, alo, ahi, w_ref, b_ref, g_ref, t_ref, olo, ohi):
    w = w_ref[...]
    ylo = hlo[...] + alo[...]
    yhi = hhi[...] + ahi[...]
    z = (jnp.dot(ylo, w[:128], preferred_element_type=_F32)
         + jnp.dot(yhi, w[128:], preferred_element_type=_F32)) + b_ref[...]
    mean = jnp.mean(z, axis=0, keepdims=True)
    zc = z - mean
    var = jnp.mean(zc * zc, axis=0, keepdims=True)
    zn = zc / jnp.sqrt(var + 1e-5) * g_ref[...] + t_ref[...]
    r = jnp.maximum(zn, 0.0)
    olo[...] = r[:, :128] + hlo[...]
    ohi[...] = r[:, 128:] + hhi[...]


def _pool_body(hlo, hhi, b_ref, wd1_ref, bd1_ref, wd2_ref, bd2_ref, out_ref):
    oh = (b_ref[...] == lax.broadcasted_iota(jnp.int32, (1, _G), 1)).astype(_F32)
    dn = (((0,), (0,)), ((), ()))
    slo = lax.dot_general(oh, hlo[...], dn, preferred_element_type=_F32)
    shi = lax.dot_general(oh, hhi[...], dn, preferred_element_type=_F32)
    cnt = lax.dot_general(oh, jnp.ones((_N, 1), _F32), dn,
                          preferred_element_type=_F32)
    inv = 1.0 / jnp.maximum(cnt, 1.0)
    wd1 = wd1_ref[...]
    t = (jnp.dot(slo * inv, wd1[:128], preferred_element_type=_F32)
         + jnp.dot(shi * inv, wd1[128:], preferred_element_type=_F32)) + bd1_ref[...]
    out_ref[...] = jnp.dot(jnp.maximum(t, 0.0), wd2_ref[...],
                           preferred_element_type=_F32) + bd2_ref[...]


def _enc_h(x, W_in, b_in):
    return pl.pallas_call(
        _enc_h_body,
        out_shape=[jax.ShapeDtypeStruct((_N, 128), _F32),
                   jax.ShapeDtypeStruct((_N, 128), _F32)],
    )(x, W_in, b_in)


def _enc_e(edge_attr, W_e, b_e):
    blk = 10000
    nblk = _E // blk
    return pl.pallas_call(
        _enc_e_body,
        grid=(nblk,),
        in_specs=[
            pl.BlockSpec((blk, 16), lambda i: (i, 0)),
            pl.BlockSpec((16, 256), lambda i: (0, 0)),
            pl.BlockSpec((1, 256), lambda i: (0, 0)),
        ],
        out_specs=[
            pl.BlockSpec((blk, 128), lambda i: (i, 0)),
            pl.BlockSpec((blk, 128), lambda i: (i, 0)),
        ],
        out_shape=[jax.ShapeDtypeStruct((_E, 128), _F32),
                   jax.ShapeDtypeStruct((_E, 128), _F32)],
    )(edge_attr, W_e, b_e)


def _layer_tc(h_lo, h_hi, agg_lo, agg_hi, W, b, gamma, beta):
    return pl.pallas_call(
        _layer_body,
        out_shape=[jax.ShapeDtypeStruct((_N, 128), _F32),
                   jax.ShapeDtypeStruct((_N, 128), _F32)],
    )(h_lo, h_hi, agg_lo, agg_hi, W, b, gamma, beta)


def _pool_tc(h_lo, h_hi, batch2, Wd1, bd1, Wd2, bd2):
    return pl.pallas_call(
        _pool_body,
        out_shape=jax.ShapeDtypeStruct((_G, 128), _F32),
    )(h_lo, h_hi, batch2, Wd1, bd1, Wd2, bd2)


def kernel(x, edge_index, edge_attr, batch, W_in, b_in, W_e, b_e,
           Ws, bs, gammas, betas, Wd1, bd1, Wd2, bd2):
    src2 = edge_index[0].reshape(_NCH, _CH)
    dst2 = edge_index[1].reshape(_NCH, _CH)
    batch2 = batch.reshape(_N, 1)
    h_lo, h_hi = _enc_h(x, W_in, b_in.reshape(1, -1))
    e_lo, e_hi = _enc_e(edge_attr, W_e, b_e.reshape(1, -1))
    for l in range(Ws.shape[0]):
        agg_lo, agg_hi = _sc_edge(h_lo, h_hi, e_lo, e_hi, src2, dst2)
        h_lo, h_hi = _layer_tc(h_lo, h_hi, agg_lo, agg_hi, Ws[l],
                               bs[l].reshape(1, -1), gammas[l].reshape(1, -1),
                               betas[l].reshape(1, -1))
    return _pool_tc(h_lo, h_hi, batch2, Wd1, bd1.reshape(1, -1),
                    Wd2, bd2.reshape(1, -1))


# speedup vs baseline: 2.8518x; 2.8518x over previous
import functools

import jax
import jax.numpy as jnp
from jax import lax
from jax.experimental import pallas as pl
from jax.experimental.pallas import tpu as pltpu
from jax.experimental.pallas import tpu_sc as plsc

_N = 10000
_E = 320000
_G = 64
_CH = 128
_NCH = _E // _CH
_NTILES = 16
_BASE_CH = _NCH // _NTILES
_EXTRA_CH = _NCH % _NTILES
_RPT = 624
_TAIL = _N - _RPT * _NTILES
_F32 = jnp.float32


def _sc_edge_body(h_lo, h_hi, e_lo, e_hi, src1, dst1, agg_lo, agg_hi,
                  acc, sidx, didx, rows, ev, sem):
    cid = lax.axis_index("c")
    sid = lax.axis_index("s")

    def zrow(r, carry):
        for c8 in range(8):
            rows[r, pl.ds(c8 * 16, 16)] = jnp.zeros((16,), _F32)
        return carry
    lax.fori_loop(0, _CH, zrow, 0)
    base0 = pl.multiple_of(sid * _RPT, 8)
    off = 0
    while off < _RPT:
        nr = min(_CH, _RPT - off)
        pltpu.sync_copy(rows.at[pl.ds(0, nr)],
                        acc.at[pl.ds(pl.multiple_of(base0 + off, 8), nr)])
        off += nr

    @pl.when(sid == _NTILES - 1)
    def _():
        pltpu.sync_copy(rows.at[pl.ds(0, _TAIL)],
                        acc.at[pl.ds(_RPT * _NTILES, _TAIL)])
    plsc.subcore_barrier()

    nch = jnp.where(sid < _EXTRA_CH, _BASE_CH + 1, _BASE_CH)

    def make_work(h_ref, e_ref):
        def work():
            def body(i, carry):
                chunk = sid + _NTILES * i
                ebase = pl.multiple_of(chunk * _CH, _CH)
                pltpu.sync_copy(src1.at[pl.ds(ebase, _CH)], sidx.at[0])
                pltpu.sync_copy(dst1.at[pl.ds(ebase, _CH)], didx.at[0])
                gat = pltpu.async_copy(h_ref.at[sidx.at[0]], rows, sem)
                pltpu.sync_copy(e_ref.at[pl.ds(ebase, _CH)], ev)
                gat.wait()

                def crow(r, cc):
                    for c8 in range(8):
                        sl = pl.ds(c8 * 16, 16)
                        ev[r, sl] = jnp.maximum(rows[r, sl] + ev[r, sl], 0.0)
                    return cc
                lax.fori_loop(0, _CH, crow, 0)
                pltpu.sync_copy(ev, acc.at[didx.at[0]], add=True)
                return carry
            lax.fori_loop(0, nch, body, 0)
        return work

    pl.when(cid == 0)(make_work(h_lo, e_lo))
    pl.when(cid == 1)(make_work(h_hi, e_hi))
    plsc.subcore_barrier()

    def make_flush(o_ref):
        def flush():
            pltpu.sync_copy(acc.at[pl.ds(base0, _RPT)],
                            o_ref.at[pl.ds(base0, _RPT)])

            @pl.when(sid == _NTILES - 1)
            def _():
                pltpu.sync_copy(acc.at[pl.ds(_RPT * _NTILES, _TAIL)],
                                o_ref.at[pl.ds(_RPT * _NTILES, _TAIL)])
        return flush

    pl.when(cid == 0)(make_flush(agg_lo))
    pl.when(cid == 1)(make_flush(agg_hi))


def _sc_edge(h_lo, h_hi, e_lo, e_hi, src1, dst1):
    f = pl.kernel(
        _sc_edge_body,
        out_type=[jax.ShapeDtypeStruct((_N, 128), _F32),
                  jax.ShapeDtypeStruct((_N, 128), _F32)],
        mesh=plsc.VectorSubcoreMesh(core_axis_name="c", subcore_axis_name="s"),
        scratch_types=[
            pltpu.VMEM_SHARED((_N, 128), _F32),
            pltpu.VMEM((1, _CH), jnp.int32),
            pltpu.VMEM((1, _CH), jnp.int32),
            pltpu.VMEM((_CH, 128), _F32),
            pltpu.VMEM((_CH, 128), _F32),
            pltpu.SemaphoreType.DMA,
        ],
    )
    return f(h_lo, h_hi, e_lo, e_hi, src1, dst1)


def _enc_h_body(x_ref, w_ref, b_ref, lo_ref, hi_ref):
    h = jnp.dot(x_ref[...], w_ref[...], preferred_element_type=_F32) + b_ref[...]
    lo_ref[...] = h[:, :128]
    hi_ref[...] = h[:, 128:]


def _enc_e_body(ea_ref, w_ref, b_ref, lo_ref, hi_ref):
    v = jnp.dot(ea_ref[...], w_ref[...], preferred_element_type=_F32) + b_ref[...]
    lo_ref[...] = v[:, :128]
    hi_ref[...] = v[:, 128:]


def _layer_body(hlo, hhi, alo, ahi, w_ref, b_ref, g_ref, t_ref, olo, ohi):
    w = w_ref[...]
    ylo = hlo[...] + alo[...]
    yhi = hhi[...] + ahi[...]
    z = (jnp.dot(ylo, w[:128], preferred_element_type=_F32)
         + jnp.dot(yhi, w[128:], preferred_element_type=_F32)) + b_ref[...]
    mean = jnp.mean(z, axis=0, keepdims=True)
    zc = z - mean
    var = jnp.mean(zc * zc, axis=0, keepdims=True)
    zn = zc / jnp.sqrt(var + 1e-5) * g_ref[...] + t_ref[...]
    r = jnp.maximum(zn, 0.0)
    olo[...] = r[:, :128] + hlo[...]
    ohi[...] = r[:, 128:] + hhi[...]


def _pool_body(hlo, hhi, b_ref, wd1_ref, bd1_ref, wd2_ref, bd2_ref, out_ref):
    oh = (b_ref[...] == lax.broadcasted_iota(jnp.int32, (1, _G), 1)).astype(_F32)
    dn = (((0,), (0,)), ((), ()))
    slo = lax.dot_general(oh, hlo[...], dn, preferred_element_type=_F32)
    shi = lax.dot_general(oh, hhi[...], dn, preferred_element_type=_F32)
    cnt = lax.dot_general(oh, jnp.ones((_N, 1), _F32), dn,
                          preferred_element_type=_F32)
    inv = 1.0 / jnp.maximum(cnt, 1.0)
    wd1 = wd1_ref[...]
    t = (jnp.dot(slo * inv, wd1[:128], preferred_element_type=_F32)
         + jnp.dot(shi * inv, wd1[128:], preferred_element_type=_F32)) + bd1_ref[...]
    out_ref[...] = jnp.dot(jnp.maximum(t, 0.0), wd2_ref[...],
                           preferred_element_type=_F32) + bd2_ref[...]


def _enc_h(x, W_in, b_in):
    return pl.pallas_call(
        _enc_h_body,
        out_shape=[jax.ShapeDtypeStruct((_N, 128), _F32),
                   jax.ShapeDtypeStruct((_N, 128), _F32)],
    )(x, W_in, b_in)


def _enc_e(edge_attr, W_e, b_e):
    blk = 10000
    nblk = _E // blk
    return pl.pallas_call(
        _enc_e_body,
        grid=(nblk,),
        in_specs=[
            pl.BlockSpec((blk, 16), lambda i: (i, 0)),
            pl.BlockSpec((16, 256), lambda i: (0, 0)),
            pl.BlockSpec((1, 256), lambda i: (0, 0)),
        ],
        out_specs=[
            pl.BlockSpec((blk, 128), lambda i: (i, 0)),
            pl.BlockSpec((blk, 128), lambda i: (i, 0)),
        ],
        out_shape=[jax.ShapeDtypeStruct((_E, 128), _F32),
                   jax.ShapeDtypeStruct((_E, 128), _F32)],
    )(edge_attr, W_e, b_e)


def _layer_tc(h_lo, h_hi, agg_lo, agg_hi, W, b, gamma, beta):
    return pl.pallas_call(
        _layer_body,
        out_shape=[jax.ShapeDtypeStruct((_N, 128), _F32),
                   jax.ShapeDtypeStruct((_N, 128), _F32)],
    )(h_lo, h_hi, agg_lo, agg_hi, W, b, gamma, beta)


def _pool_tc(h_lo, h_hi, batch2, Wd1, bd1, Wd2, bd2):
    return pl.pallas_call(
        _pool_body,
        out_shape=jax.ShapeDtypeStruct((_G, 128), _F32),
    )(h_lo, h_hi, batch2, Wd1, bd1, Wd2, bd2)


def kernel(x, edge_index, edge_attr, batch, W_in, b_in, W_e, b_e,
           Ws, bs, gammas, betas, Wd1, bd1, Wd2, bd2):
    src1 = edge_index[0]
    dst1 = edge_index[1]
    batch2 = batch.reshape(_N, 1)
    h_lo, h_hi = _enc_h(x, W_in, b_in.reshape(1, -1))
    e_lo, e_hi = _enc_e(edge_attr, W_e, b_e.reshape(1, -1))
    for l in range(Ws.shape[0]):
        agg_lo, agg_hi = _sc_edge(h_lo, h_hi, e_lo, e_hi, src1, dst1)
        h_lo, h_hi = _layer_tc(h_lo, h_hi, agg_lo, agg_hi, Ws[l],
                               bs[l].reshape(1, -1), gammas[l].reshape(1, -1),
                               betas[l].reshape(1, -1))
    return _pool_tc(h_lo, h_hi, batch2, Wd1, bd1.reshape(1, -1),
                    Wd2, bd2.reshape(1, -1))
